# baseline (speedup 1.0000x reference)
import functools
import math
import jax
import jax.numpy as jnp
from jax import lax
from jax.experimental import pallas as pl
from jax.experimental.pallas import tpu as pltpu
from jax.experimental.pallas import tpu_sc as plsc

INPUT_SIZE = 128
CONTEXT_SIZE = 256
CONTEXT_MAP_SIZE = 16
BATCH = 4096
PRED_CLIPPING = 0.01

_NUM_CORES = 2
_NUM_SUBCORES = 16
_NW = _NUM_CORES * _NUM_SUBCORES
_BPW = BATCH // _NW
_LANES = 16
_CHUNKS = INPUT_SIZE // _LANES
_NSTREAM = 2
_SPB = _BPW // _NSTREAM

_CLIP_LO = float(math.log(PRED_CLIPPING) - math.log1p(-PRED_CLIPPING))
_CLIP_HI = float(math.log(1.0 - PRED_CLIPPING) - math.log(PRED_CLIPPING))


def _bf16_bits(x):
    b = jax.lax.bitcast_convert_type(x, jnp.uint32)
    r = b + jnp.uint32(0x7FFF) + ((b >> jnp.uint32(16)) & jnp.uint32(1))
    return r >> jnp.uint32(16)


def _idx_body(cm_ref, ci_ref, cb_ref, l_ref, idx_ref, ltp_ref):
    d = jnp.dot(cm_ref[...], ci_ref[...], preferred_element_type=jnp.float32)
    acc = jnp.zeros((1, BATCH), jnp.float32)
    for i in range(CONTEXT_MAP_SIZE):
        bit = (d[i : i + 1, :] > cb_ref[i]).astype(jnp.float32)
        acc = acc + bit * jnp.float32(2.0 ** i)
    idx_ref[...] = acc.astype(jnp.int32)
    l = l_ref[...]
    packed = _bf16_bits(l[: INPUT_SIZE // 2, :]) | (
        _bf16_bits(l[INPUT_SIZE // 2 :, :]) << jnp.uint32(16)
    )
    ltp_ref[...] = jax.lax.bitcast_convert_type(packed, jnp.int32).T


@functools.lru_cache(maxsize=1)
def _build_sc_stage():
    mesh = plsc.VectorSubcoreMesh(core_axis_name="c", subcore_axis_name="s")

    @functools.partial(
        pl.kernel,
        mesh=mesh,
        out_type=jax.ShapeDtypeStruct((BATCH,), jnp.float32),
        scratch_types=[
            pltpu.VMEM((_BPW,), jnp.int32),
            pltpu.VMEM((_BPW, INPUT_SIZE), jnp.float32),
            pltpu.VMEM((_BPW, INPUT_SIZE // 2), jnp.int32),
            pltpu.VMEM((_BPW, _LANES), jnp.float32),
            pltpu.VMEM((_BPW,), jnp.float32),
            pltpu.SemaphoreType.DMA,
            pltpu.SemaphoreType.DMA,
        ],
    )
    def _sc_stage(idx_hbm, table_hbm, lt_hbm, out_hbm,
                  idx_v, rows_v, lt_v, tmp_v, out_v, gsem, lsem):
        wid = lax.axis_index("s") * _NUM_CORES + lax.axis_index("c")
        base = wid * _BPW
        lane = lax.iota(jnp.int32, _LANES)
        pltpu.sync_copy(idx_hbm.at[pl.ds(base, _BPW)], idx_v)
        copies = []
        for s in range(_NSTREAM):
            lo = s * _SPB
            g = pltpu.async_copy(
                table_hbm.at[idx_v.at[pl.ds(lo, _SPB)]],
                rows_v.at[pl.ds(lo, _SPB)],
                gsem,
            )
            lc = pltpu.async_copy(
                lt_hbm.at[pl.ds(base + lo, _SPB)],
                lt_v.at[pl.ds(lo, _SPB)],
                lsem,
            )
            copies.append((g, lc))

        def _hsum_all_lanes(v):
            for sh in (8, 4, 2, 1):
                v = v + jnp.take_along_axis(v, lane ^ sh, axis=0)
            return v

        def gbody(jg, carry):
            j0 = jg * _LANES
            for l in range(_LANES):
                j = j0 + l
                acc = None
                for t in range(_CHUNKS // 2):
                    v = lt_v[j, pl.ds(t * _LANES, _LANES)]
                    lo_f = jax.lax.bitcast_convert_type(
                        v << 16, jnp.float32)
                    hi_f = jax.lax.bitcast_convert_type(
                        v & jnp.int32(-65536), jnp.float32)
                    term = (rows_v[j, pl.ds(t * _LANES, _LANES)] * lo_f
                            + rows_v[j, pl.ds(INPUT_SIZE // 2 + t * _LANES,
                                              _LANES)] * hi_f)
                    acc = term if acc is None else acc + term
                tmp_v[j, :] = _hsum_all_lanes(acc)
            outacc = jnp.zeros((_LANES,), jnp.float32)
            for l in range(_LANES):
                outacc = jnp.where(lane == l, tmp_v[j0 + l, :], outacc)
            out_v[pl.ds(j0, _LANES)] = jnp.clip(
                outacc, jnp.float32(_CLIP_LO), jnp.float32(_CLIP_HI))
            return carry

        groups_per_stream = _SPB // _LANES
        for s, (g, lc) in enumerate(copies):
            g.wait()
            lc.wait()
            lax.fori_loop(s * groups_per_stream, (s + 1) * groups_per_stream,
                          gbody, 0)
        pltpu.sync_copy(out_v, out_hbm.at[pl.ds(base, _BPW)])

    return _sc_stage


def kernel(logits, context_inputs, context_maps, context_bias, weights, boolean_converter):
    del boolean_converter
    cb = context_bias.reshape(CONTEXT_MAP_SIZE)
    idx2d, logits_tp = pl.pallas_call(
        _idx_body,
        in_specs=[
            pl.BlockSpec(memory_space=pltpu.VMEM),
            pl.BlockSpec(memory_space=pltpu.VMEM),
            pl.BlockSpec(memory_space=pltpu.SMEM),
            pl.BlockSpec(memory_space=pltpu.VMEM),
        ],
        out_shape=(
            jax.ShapeDtypeStruct((1, BATCH), jnp.int32),
            jax.ShapeDtypeStruct((BATCH, INPUT_SIZE // 2), jnp.int32),
        ),
    )(context_maps, context_inputs, cb, logits)
    idx = idx2d.reshape(BATCH)

    return _build_sc_stage()(idx, weights, logits_tp)


# speedup vs baseline: 1.0146x; 1.0146x over previous
import functools
import math
import jax
import jax.numpy as jnp
from jax import lax
from jax.experimental import pallas as pl
from jax.experimental.pallas import tpu as pltpu
from jax.experimental.pallas import tpu_sc as plsc

INPUT_SIZE = 128
CONTEXT_SIZE = 256
CONTEXT_MAP_SIZE = 16
BATCH = 4096
PRED_CLIPPING = 0.01

_NUM_CORES = 2
_NUM_SUBCORES = 16
_NW = _NUM_CORES * _NUM_SUBCORES
_BPW = BATCH // _NW
_LANES = 16
_CHUNKS = INPUT_SIZE // _LANES
_NSTREAM = 4
_SPB = _BPW // _NSTREAM

_CLIP_LO = float(math.log(PRED_CLIPPING) - math.log1p(-PRED_CLIPPING))
_CLIP_HI = float(math.log(1.0 - PRED_CLIPPING) - math.log(PRED_CLIPPING))


def _bf16_bits(x):
    b = jax.lax.bitcast_convert_type(x, jnp.uint32)
    r = b + jnp.uint32(0x7FFF) + ((b >> jnp.uint32(16)) & jnp.uint32(1))
    return r >> jnp.uint32(16)


def _idx_body(cm_ref, ci_ref, cb_ref, l_ref, idx_ref, ltp_ref):
    d = jnp.dot(cm_ref[...], ci_ref[...], preferred_element_type=jnp.float32)
    acc = jnp.zeros((1, BATCH), jnp.float32)
    for i in range(CONTEXT_MAP_SIZE):
        bit = (d[i : i + 1, :] > cb_ref[i]).astype(jnp.float32)
        acc = acc + bit * jnp.float32(2.0 ** i)
    idx_ref[...] = acc.astype(jnp.int32)
    l = l_ref[...]
    packed = _bf16_bits(l[: INPUT_SIZE // 2, :]) | (
        _bf16_bits(l[INPUT_SIZE // 2 :, :]) << jnp.uint32(16)
    )
    ltp_ref[...] = jax.lax.bitcast_convert_type(packed, jnp.int32).T


@functools.lru_cache(maxsize=1)
def _build_sc_stage():
    mesh = plsc.VectorSubcoreMesh(core_axis_name="c", subcore_axis_name="s")

    @functools.partial(
        pl.kernel,
        mesh=mesh,
        out_type=jax.ShapeDtypeStruct((BATCH,), jnp.float32),
        scratch_types=[
            pltpu.VMEM((_BPW,), jnp.int32),
            pltpu.VMEM((_BPW, INPUT_SIZE), jnp.float32),
            pltpu.VMEM((_BPW, INPUT_SIZE // 2), jnp.int32),
            pltpu.VMEM((_BPW,), jnp.float32),
            pltpu.SemaphoreType.DMA,
            pltpu.SemaphoreType.DMA,
        ],
    )
    def _sc_stage(idx_hbm, table_hbm, lt_hbm, out_hbm,
                  idx_v, rows_v, lt_v, out_v, gsem, lsem):
        wid = lax.axis_index("s") * _NUM_CORES + lax.axis_index("c")
        base = wid * _BPW
        lane = lax.iota(jnp.int32, _LANES)
        pltpu.sync_copy(idx_hbm.at[pl.ds(base, _BPW)], idx_v)
        copies = []
        for s in range(_NSTREAM):
            lo = s * _SPB
            g = pltpu.async_copy(
                table_hbm.at[idx_v.at[pl.ds(lo, _SPB)]],
                rows_v.at[pl.ds(lo, _SPB)],
                gsem,
            )
            lc = pltpu.async_copy(
                lt_hbm.at[pl.ds(base + lo, _SPB)],
                lt_v.at[pl.ds(lo, _SPB)],
                lsem,
            )
            copies.append((g, lc))

        def _merge(a, b, k):
            m = (lane & k) == 0
            return jnp.where(m, a, b) + jnp.take_along_axis(
                jnp.where(m, b, a), lane ^ k, axis=0)

        def gbody(jg, carry):
            j0 = jg * _LANES
            stack = []
            for l in range(_LANES):
                j = j0 + l
                acc = None
                for t in range(_CHUNKS // 2):
                    v = lt_v[j, pl.ds(t * _LANES, _LANES)]
                    lo_f = jax.lax.bitcast_convert_type(
                        v << 16, jnp.float32)
                    hi_f = jax.lax.bitcast_convert_type(
                        v & jnp.int32(-65536), jnp.float32)
                    term = (rows_v[j, pl.ds(t * _LANES, _LANES)] * lo_f
                            + rows_v[j, pl.ds(INPUT_SIZE // 2 + t * _LANES,
                                              _LANES)] * hi_f)
                    acc = term if acc is None else acc + term
                node, level = acc, 0
                while stack and stack[-1][1] == level:
                    prev, _ = stack.pop()
                    node = _merge(prev, node, 1 << level)
                    level += 1
                stack.append((node, level))
            out_v[pl.ds(j0, _LANES)] = jnp.clip(
                stack[0][0], jnp.float32(_CLIP_LO), jnp.float32(_CLIP_HI))
            return carry

        groups_per_stream = _SPB // _LANES
        for s, (g, lc) in enumerate(copies):
            g.wait()
            lc.wait()
            lax.fori_loop(s * groups_per_stream, (s + 1) * groups_per_stream,
                          gbody, 0)
        pltpu.sync_copy(out_v, out_hbm.at[pl.ds(base, _BPW)])

    return _sc_stage


def kernel(logits, context_inputs, context_maps, context_bias, weights, boolean_converter):
    del boolean_converter
    cb = context_bias.reshape(CONTEXT_MAP_SIZE)
    idx2d, logits_tp = pl.pallas_call(
        _idx_body,
        in_specs=[
            pl.BlockSpec(memory_space=pltpu.VMEM),
            pl.BlockSpec(memory_space=pltpu.VMEM),
            pl.BlockSpec(memory_space=pltpu.SMEM),
            pl.BlockSpec(memory_space=pltpu.VMEM),
        ],
        out_shape=(
            jax.ShapeDtypeStruct((1, BATCH), jnp.int32),
            jax.ShapeDtypeStruct((BATCH, INPUT_SIZE // 2), jnp.int32),
        ),
    )(context_maps, context_inputs, cb, logits)
    idx = idx2d.reshape(BATCH)

    return _build_sc_stage()(idx, weights, logits_tp)
